# baseline (device time: 11884 ns/iter reference)
import jax
import jax.numpy as jnp
from jax import lax
from jax.experimental import pallas as pl
from jax.experimental.pallas import tpu as pltpu

N_DEV = 16
EPS = 1e-5

import os
import contextlib
_COMM_MODE = os.environ.get("COMM_MODE", "full")
if os.environ.get("SKIP_COMM", "0") == "1":
    _COMM_MODE = "none"
_SKIP_COMM = _COMM_MODE == "none"
_DO_RDMA = _COMM_MODE in ("full", "tiny")
_SLOT_N = 128 if _COMM_MODE == "tiny" else 512
_SCOPES = os.environ.get("KERNEL_SCOPES", "0") == "1"


def _scope(name):
    return jax.named_scope(name) if _SCOPES else contextlib.nullcontext()


def kernel(x, gamma, beta):
    m, n_loc = x.shape
    n_total = n_loc * N_DEV

    def body(x_ref, g_ref, b_ref, out_ref, comm_ref, send_sems, recv_sems):
        my = lax.axis_index("i")

        if not _SKIP_COMM:
            with _scope("barrier_signal"):
                barrier_sem = pltpu.get_barrier_semaphore()
                for d in range(1, N_DEV):
                    pl.semaphore_signal(
                        barrier_sem, inc=1,
                        device_id=((my + d) % N_DEV,),
                        device_id_type=pl.DeviceIdType.MESH,
                    )

        with _scope("stats"):
            xv = x_ref[:, :].astype(jnp.float32)
            s1 = jnp.sum(xv, axis=1)
            s2 = jnp.sum(xv * xv, axis=1)
            stats = jnp.stack([s1, s2], axis=0)
            comm_ref[0, :, :] = stats[:, :_SLOT_N]

        rdmas = []
        if not _SKIP_COMM:
            with _scope("barrier_wait"):
                pl.semaphore_wait(barrier_sem, N_DEV - 1)

        if _DO_RDMA:
            with _scope("send_issue"):
                for d in range(1, N_DEV):
                    rdma = pltpu.make_async_remote_copy(
                        src_ref=comm_ref.at[0],
                        dst_ref=comm_ref.at[d],
                        send_sem=send_sems.at[d],
                        recv_sem=recv_sems.at[d],
                        device_id=((my + d) % N_DEV,),
                        device_id_type=pl.DeviceIdType.MESH,
                    )
                    rdma.start()
                    rdmas.append(rdma)
            with _scope("recv_wait"):
                for rdma in rdmas:
                    rdma.wait_recv()

        with _scope("reduce"):
            if _COMM_MODE != "full":
                tot = stats * float(N_DEV)
            else:
                tot = jnp.sum(comm_ref[:, :, :], axis=0)
            mean_p = tot[0:1, :] / n_total
            ex2_p = tot[1:2, :] / n_total
            rstd_p = lax.rsqrt(ex2_p - mean_p * mean_p + EPS)
            mean = mean_p.reshape(m, 1)
            rstd = rstd_p.reshape(m, 1)
        with _scope("normalize"):
            g = g_ref[:].reshape(1, n_loc)
            b = b_ref[:].reshape(1, n_loc)
            out = (xv - mean) * rstd * g + b
            out_ref[:, :] = out.astype(out_ref.dtype)

        with _scope("send_wait"):
            for rdma in rdmas:
                rdma.wait_send()

    return pl.pallas_call(
        body,
        out_shape=jax.ShapeDtypeStruct((m, n_loc), jnp.float32),
        in_specs=[
            pl.BlockSpec(memory_space=pltpu.VMEM),
            pl.BlockSpec(memory_space=pltpu.VMEM),
            pl.BlockSpec(memory_space=pltpu.VMEM),
        ],
        out_specs=pl.BlockSpec(memory_space=pltpu.VMEM),
        scratch_shapes=[
            pltpu.VMEM((N_DEV, 2, _SLOT_N), jnp.float32),
            pltpu.SemaphoreType.DMA((N_DEV,)),
            pltpu.SemaphoreType.DMA((N_DEV,)),
        ],
        compiler_params=pltpu.CompilerParams(
            collective_id=None if _SKIP_COMM else 0
        ),
    )(x, gamma, beta)


# device time: 11179 ns/iter; 1.0631x vs baseline; 1.0631x over previous
import jax
import jax.numpy as jnp
from jax import lax
from jax.experimental import pallas as pl
from jax.experimental.pallas import tpu as pltpu

N_DEV = 16
EPS = 1e-5

import os
import contextlib
_COMM_MODE = os.environ.get("COMM_MODE", "full")
if os.environ.get("SKIP_COMM", "0") == "1":
    _COMM_MODE = "none"
_SKIP_COMM = _COMM_MODE == "none"
_DO_RDMA = _COMM_MODE in ("full", "tiny")
_SLOT_N = 128 if _COMM_MODE == "tiny" else 512
_SCOPES = os.environ.get("KERNEL_SCOPES", "0") == "1"


def _scope(name):
    return jax.named_scope(name) if _SCOPES else contextlib.nullcontext()


def kernel(x, gamma, beta):
    m, n_loc = x.shape
    n_total = n_loc * N_DEV

    def body(x_ref, g_ref, b_ref, out_ref, comm_ref, send_sems, recv_sems):
        my = lax.axis_index("i")

        if not _SKIP_COMM:
            with _scope("barrier_signal"):
                barrier_sem = pltpu.get_barrier_semaphore()
                for d in range(1, N_DEV):
                    pl.semaphore_signal(
                        barrier_sem, inc=1,
                        device_id=((my + d) % N_DEV,),
                        device_id_type=pl.DeviceIdType.MESH,
                    )

        with _scope("stats"):
            xv = x_ref[:, :].astype(jnp.float32)
            s1 = jnp.sum(xv, axis=1)
            s2 = jnp.sum(xv * xv, axis=1)
            stats = jnp.stack([s1, s2], axis=0)
            comm_ref[0, :, :] = stats[:, :_SLOT_N]

        rdmas = []
        if not _SKIP_COMM:
            with _scope("barrier_wait"):
                pl.semaphore_wait(barrier_sem, N_DEV - 1)

        if _DO_RDMA:
            with _scope("send_issue"):
                for d in range(1, N_DEV):
                    rdma = pltpu.make_async_remote_copy(
                        src_ref=comm_ref.at[0],
                        dst_ref=comm_ref.at[d],
                        send_sem=send_sems.at[d],
                        recv_sem=recv_sems.at[d],
                        device_id=((my + d) % N_DEV,),
                        device_id_type=pl.DeviceIdType.MESH,
                    )
                    rdma.start()
                    rdmas.append(rdma)
            with _scope("recv_wait"):
                for rdma in rdmas:
                    rdma.wait_recv()

        with _scope("reduce"):
            if _COMM_MODE != "full":
                tot = stats * float(N_DEV)
            else:
                tot = jnp.sum(comm_ref[:, :, :], axis=0)
            mean_p = tot[0:1, :] / n_total
            ex2_p = tot[1:2, :] / n_total
            rstd_p = lax.rsqrt(ex2_p - mean_p * mean_p + EPS)
            mean = mean_p.reshape(m, 1)
            rstd = rstd_p.reshape(m, 1)
        with _scope("normalize"):
            out = (xv - mean) * rstd * g_ref[:, :] + b_ref[:, :]
            out_ref[:, :] = out.astype(out_ref.dtype)

        with _scope("send_wait"):
            for rdma in rdmas:
                rdma.wait_send()

    return pl.pallas_call(
        body,
        out_shape=jax.ShapeDtypeStruct((m, n_loc), jnp.float32),
        in_specs=[
            pl.BlockSpec(memory_space=pltpu.VMEM),
            pl.BlockSpec(memory_space=pltpu.VMEM),
            pl.BlockSpec(memory_space=pltpu.VMEM),
        ],
        out_specs=pl.BlockSpec(memory_space=pltpu.VMEM),
        scratch_shapes=[
            pltpu.VMEM((N_DEV, 2, _SLOT_N), jnp.float32),
            pltpu.SemaphoreType.DMA((N_DEV,)),
            pltpu.SemaphoreType.DMA((N_DEV,)),
        ],
        compiler_params=pltpu.CompilerParams(
            collective_id=None if _SKIP_COMM else 0
        ),
    )(x, gamma.reshape(1, -1), beta.reshape(1, -1))
